# baseline (speedup 1.0000x reference)
import functools

import jax
import jax.numpy as jnp
from jax import lax
from jax.experimental import pallas as pl
from jax.experimental.pallas import tpu as pltpu
from jax.experimental.pallas import tpu_sc as plsc

N = 10000
E = 320000
EL = 100000
D_IN = 384
H = 128

NC = 2
NS = 16
NW = NC * NS

CHUNK = 128
EPW = 10240
EP = NW * EPW
NCHUNK = EPW // CHUNK
KSLOW_CORE = 0
KSLOW = 36
KFAST = 2 * NCHUNK - KSLOW
NA = 10240
STRIPE = NA // NS

ELW = 3328
ELP = NW * ELW
NLCHUNK = ELW // CHUNK

_mesh = plsc.VectorSubcoreMesh(core_axis_name="c", subcore_axis_name="s")


def _dg(a, b):
    return lax.dot_general(a, b, (((1,), (1,)), ((), ())),
                           preferred_element_type=jnp.float32)


def _tc1_body(x_ref, wl_ref, bl_ref, w1l_ref, b1l_ref, w1r_ref,
              t1_ref, r1_ref):
    h0 = _dg(x_ref[...], wl_ref[...]) + bl_ref[...]
    t1_ref[...] = _dg(h0, w1l_ref[...])
    r1_ref[...] = _dg(h0, w1r_ref[...]) + b1l_ref[...]


def _tc1(x, W_lin, b_lin, W1l, b1l, W1r):
    R = 2000
    grid = (N // R,)
    full = lambda shape: pl.BlockSpec(shape, lambda i: (0,) * len(shape))
    return pl.pallas_call(
        _tc1_body,
        grid=grid,
        in_specs=[
            pl.BlockSpec((R, D_IN), lambda i: (i, 0)),
            full((H, D_IN)), full((1, H)), full((H, H)), full((1, H)),
            full((H, H)),
        ],
        out_specs=[pl.BlockSpec((R, H), lambda i: (i, 0))] * 2,
        out_shape=[jax.ShapeDtypeStruct((N, H), jnp.float32)] * 2,
    )(x, W_lin, b_lin.reshape(1, H), W1l, b1l.reshape(1, H), W1r)


def _tc2_body(aggp_ref, cntp_ref, r1_ref, w2l_ref, b2l_ref, w2r_ref,
              t2_ref, r2_ref):
    agg = aggp_ref[0] + aggp_ref[1]
    cnt = cntp_ref[0, :, 0:1] + cntp_ref[1, :, 0:1]
    inv = 1.0 / jnp.maximum(cnt, 1.0)
    h1 = jnp.maximum(agg * inv + r1_ref[...], 0.0)
    t2_ref[...] = _dg(h1, w2l_ref[...])
    r2_ref[...] = _dg(h1, w2r_ref[...]) + b2l_ref[...]


def _tc2(aggp, cntp, r1, W2l, b2l, W2r):
    R = 2000
    grid = (N // R,)
    full = lambda shape: pl.BlockSpec(shape, lambda i: (0,) * len(shape))
    return pl.pallas_call(
        _tc2_body,
        grid=grid,
        in_specs=[
            pl.BlockSpec((NC, R, H), lambda i: (0, i, 0)),
            pl.BlockSpec((NC, R, H), lambda i: (0, i, 0)),
            pl.BlockSpec((R, H), lambda i: (i, 0)),
            full((H, H)), full((1, H)), full((H, H)),
        ],
        out_specs=[pl.BlockSpec((R, H), lambda i: (i, 0))] * 2,
        out_shape=[jax.ShapeDtypeStruct((N, H), jnp.float32)] * 2,
    )(aggp, cntp, r1, W2l, b2l.reshape(1, H), W2r)


def _tc3_body(aggp_ref, cntp_ref, r2_ref, h2_ref):
    agg = aggp_ref[0] + aggp_ref[1]
    cnt = cntp_ref[0, :, 0:1] + cntp_ref[1, :, 0:1]
    inv = 1.0 / jnp.maximum(cnt, 1.0)
    h2_ref[...] = agg * inv + r2_ref[...]


def _tc3(aggp, cntp, r2):
    R = 2048
    grid = (NA // R,)
    return pl.pallas_call(
        _tc3_body,
        grid=grid,
        in_specs=[
            pl.BlockSpec((NC, R, H), lambda i: (0, i, 0)),
            pl.BlockSpec((NC, R, H), lambda i: (0, i, 0)),
            pl.BlockSpec((R, H), lambda i: (i, 0)),
        ],
        out_specs=pl.BlockSpec((R, H), lambda i: (i, 0)),
        out_shape=jax.ShapeDtypeStruct((NA, H), jnp.float32),
    )(aggp, cntp, r2)


def _seg_body(tbl, srcp, dstp, zh, aggp,
              sv0, sv1, sv2, sv3, dv0, dv1, dv2, dv3, rows0, rows1, aggs,
              si0, si1, si2, si3, sg0, sg1, ss0, ss1):
    c = lax.axis_index("c")
    s = lax.axis_index("s")
    srcv = [sv0, sv1, sv2, sv3]
    dstv = [dv0, dv1, dv2, dv3]
    rows = [rows0, rows1]
    sem_i = [si0, si1, si2, si3]
    sem_g = [sg0, sg1]
    sem_s = [ss0, ss1]

    r0 = s * STRIPE
    pltpu.sync_copy(zh.at[pl.ds(r0, STRIPE)], aggs.at[pl.ds(r0, STRIPE)])
    plsc.subcore_barrier()
    K = jnp.where(c == KSLOW_CORE, KSLOW, KFAST)
    nquad = jnp.where(c == KSLOW_CORE, KSLOW // 4, KFAST // 4)
    base_chunk = jnp.where(
        c == KSLOW_CORE,
        jnp.where(c == 0, s * KSLOW, NS * KFAST + s * KSLOW),
        jnp.where(c == 0, s * KFAST, NS * KSLOW + s * KFAST))
    base = base_chunk * CHUNK

    def issue_idx(k, q):
        pltpu.async_copy(srcp.at[pl.ds(base + k * CHUNK, CHUNK)],
                         srcv[q], sem_i[q])
        pltpu.async_copy(dstp.at[pl.ds(base + k * CHUNK, CHUNK)],
                         dstv[q], sem_i[q])

    def wait_idx(q):
        pltpu.make_async_copy(srcp.at[pl.ds(base, CHUNK)],
                              srcv[q], sem_i[q]).wait()
        pltpu.make_async_copy(dstp.at[pl.ds(base, CHUNK)],
                              dstv[q], sem_i[q]).wait()

    def wait_scat(b, q):
        pltpu.make_async_copy(rows[b], aggs.at[dstv[q]], sem_s[b]).wait()

    issue_idx(0, 0)
    issue_idx(1, 1)

    def quad(gg, _):
        for j in range(4):
            k = gg * 4 + j
            b = j % 2
            wait_idx(j)

            @pl.when(k >= 2)
            def _():
                wait_scat(b, (j + 2) % 4)

            @pl.when(k < K - 2)
            def _():
                issue_idx(k + 2, (j + 2) % 4)

            pltpu.async_copy(tbl.at[srcv[j]], rows[b], sem_g[b])
            pltpu.make_async_copy(tbl.at[srcv[j]], rows[b], sem_g[b]).wait()
            pltpu.async_copy(rows[b], aggs.at[dstv[j]], sem_s[b], add=True)
        return 0
    lax.fori_loop(0, nquad, quad, 0)
    wait_scat(0, 2)
    wait_scat(1, 3)
    plsc.subcore_barrier()

    pltpu.sync_copy(aggs.at[pl.ds(r0, STRIPE)], aggp.at[c, pl.ds(r0, STRIPE)])


_seg = pl.kernel(
    _seg_body,
    out_type=jax.ShapeDtypeStruct((NC, NA, H), jnp.float32),
    mesh=_mesh,
    scratch_types=(
        [pltpu.VMEM((CHUNK,), jnp.int32)] * 8 +
        [pltpu.VMEM((CHUNK, H), jnp.float32)] * 2 +
        [pltpu.VMEM_SHARED((NA, H), jnp.float32)] +
        [pltpu.SemaphoreType.DMA] * 8
    ),
)


def _cnt_body(dstp, zh, cntp, dstv, ones, cnts):
    c = lax.axis_index("c")
    s = lax.axis_index("s")
    wid = c * NS + s

    r0 = s * STRIPE
    pltpu.sync_copy(zh.at[pl.ds(r0, STRIPE)], cnts.at[pl.ds(r0, STRIPE)])

    def fill1(i, _):
        for j in range(H // 16):
            ones[i, pl.ds(j * 16, 16)] = jnp.ones((16,), jnp.float32)
        return 0
    lax.fori_loop(0, CHUNK, fill1, 0)
    plsc.subcore_barrier()
    base = wid * EPW

    def chunk(k, _):
        pltpu.sync_copy(dstp.at[pl.ds(base + k * CHUNK, CHUNK)], dstv)
        pltpu.sync_copy(ones, cnts.at[dstv], add=True)
        return 0
    lax.fori_loop(0, NCHUNK, chunk, 0)
    plsc.subcore_barrier()
    pltpu.sync_copy(cnts.at[pl.ds(r0, STRIPE)], cntp.at[c, pl.ds(r0, STRIPE)])


_cnt = pl.kernel(
    _cnt_body,
    out_type=jax.ShapeDtypeStruct((NC, NA, H), jnp.float32),
    mesh=_mesh,
    scratch_types=[
        pltpu.VMEM((CHUNK,), jnp.int32),
        pltpu.VMEM((CHUNK, H), jnp.float32),
        pltpu.VMEM_SHARED((NA, H), jnp.float32),
    ],
)


def _cls_body(tbl, headp, tailp, outp, hidx, tidx,
              hrows, trows, outv, tbls, sem_h, sem_t):
    c = lax.axis_index("c")
    s = lax.axis_index("s")
    wid = c * NS + s
    r0 = s * STRIPE
    pltpu.sync_copy(tbl.at[pl.ds(r0, STRIPE)], tbls.at[pl.ds(r0, STRIPE)])
    pltpu.sync_copy(headp.at[pl.ds(wid * ELW, ELW)], hidx)
    pltpu.sync_copy(tailp.at[pl.ds(wid * ELW, ELW)], tidx)
    plsc.subcore_barrier()

    lanes = lax.iota(jnp.int32, 16)

    def chunk(k, _):
        pltpu.async_copy(tbls.at[hidx.at[pl.ds(k * CHUNK, CHUNK)]],
                         hrows, sem_h)
        pltpu.async_copy(tbls.at[tidx.at[pl.ds(k * CHUNK, CHUNK)]],
                         trows, sem_t)
        pltpu.make_async_copy(tbls.at[hidx.at[pl.ds(0, CHUNK)]],
                              hrows, sem_h).wait()
        pltpu.make_async_copy(tbls.at[tidx.at[pl.ds(0, CHUNK)]],
                              trows, sem_t).wait()

        def group(g, _):
            def edge(i, res):
                e = g * 16 + i
                acc = hrows[e, pl.ds(0, 16)] * trows[e, pl.ds(0, 16)]
                for j in range(1, H // 16):
                    acc = acc + (hrows[e, pl.ds(j * 16, 16)] *
                                 trows[e, pl.ds(j * 16, 16)])
                return jnp.where(lanes == i, jnp.sum(acc), res)
            res = lax.fori_loop(0, 16, edge, jnp.zeros((16,), jnp.float32))
            outv[pl.ds(k * CHUNK + g * 16, 16)] = res
            return 0
        lax.fori_loop(0, CHUNK // 16, group, 0)
        return 0
    lax.fori_loop(0, NLCHUNK, chunk, 0)
    pltpu.sync_copy(outv, outp.at[pl.ds(wid * ELW, ELW)])


_cls = pl.kernel(
    _cls_body,
    out_type=jax.ShapeDtypeStruct((ELP,), jnp.float32),
    mesh=_mesh,
    compiler_params=pltpu.CompilerParams(needs_layout_passes=False),
    scratch_types=(
        [pltpu.VMEM((ELW,), jnp.int32)] * 2 +
        [pltpu.VMEM((CHUNK, H), jnp.float32)] * 2 +
        [pltpu.VMEM((ELW,), jnp.float32)] +
        [pltpu.VMEM_SHARED((NA, H), jnp.float32)] +
        [pltpu.SemaphoreType.DMA] * 2
    ),
)


def kernel(x, edge_index, edge_label_index, W_lin, b_lin,
           W1l, b1l, W1r, W2l, b2l, W2r):
    src = jnp.concatenate([edge_index[0], jnp.zeros((EP - E,), jnp.int32)])
    dst = jnp.concatenate([edge_index[1], jnp.full((EP - E,), N, jnp.int32)])
    head = jnp.concatenate(
        [edge_label_index[0], jnp.zeros((ELP - EL,), jnp.int32)])
    tail = jnp.concatenate(
        [edge_label_index[1], jnp.zeros((ELP - EL,), jnp.int32)])
    zh = jnp.zeros((NA, H), jnp.float32)

    cntp = _cnt(dst, zh)
    t1, r1 = _tc1(x, W_lin, b_lin, W1l, b1l, W1r)
    aggp1 = _seg(t1, src, dst, zh)
    t2, r2 = _tc2(aggp1, cntp, r1, W2l, b2l, W2r)
    aggp2 = _seg(t2, src, dst, zh)
    h2 = _tc3(aggp2, cntp, r2)
    return _cls(h2, head, tail)[:EL]


# speedup vs baseline: 1.1400x; 1.1400x over previous
import functools

import jax
import jax.numpy as jnp
from jax import lax
from jax.experimental import pallas as pl
from jax.experimental.pallas import tpu as pltpu
from jax.experimental.pallas import tpu_sc as plsc

N = 10000
E = 320000
EL = 100000
D_IN = 384
H = 128

NC = 2
NS = 16
NW = NC * NS

CHUNK = 128
EPW = 10240
EP = NW * EPW
NCHUNK = EPW // CHUNK
KSLOW_CORE = 1
KSLOW = 48
KFAST = 2 * NCHUNK - KSLOW
NA = 10240
STRIPE = NA // NS

ELW = 3328
ELP = NW * ELW
NLCHUNK = ELW // CHUNK

_mesh = plsc.VectorSubcoreMesh(core_axis_name="c", subcore_axis_name="s")


def _dg(a, b):
    return lax.dot_general(a, b, (((1,), (1,)), ((), ())),
                           preferred_element_type=jnp.float32)


def _tc1_body(x_ref, wl_ref, bl_ref, w1l_ref, b1l_ref, w1r_ref,
              t1_ref, r1_ref):
    h0 = _dg(x_ref[...], wl_ref[...]) + bl_ref[...]
    t1_ref[...] = _dg(h0, w1l_ref[...])
    r1_ref[...] = _dg(h0, w1r_ref[...]) + b1l_ref[...]


def _tc1(x, W_lin, b_lin, W1l, b1l, W1r):
    R = 2000
    grid = (N // R,)
    full = lambda shape: pl.BlockSpec(shape, lambda i: (0,) * len(shape))
    return pl.pallas_call(
        _tc1_body,
        grid=grid,
        in_specs=[
            pl.BlockSpec((R, D_IN), lambda i: (i, 0)),
            full((H, D_IN)), full((1, H)), full((H, H)), full((1, H)),
            full((H, H)),
        ],
        out_specs=[pl.BlockSpec((R, H), lambda i: (i, 0))] * 2,
        out_shape=[jax.ShapeDtypeStruct((N, H), jnp.float32)] * 2,
    )(x, W_lin, b_lin.reshape(1, H), W1l, b1l.reshape(1, H), W1r)


def _tc2_body(aggp_ref, cntp_ref, r1_ref, w2l_ref, b2l_ref, w2r_ref,
              t2_ref, r2_ref):
    agg = aggp_ref[0] + aggp_ref[1]
    cnt = cntp_ref[0, :, 0:1] + cntp_ref[1, :, 0:1]
    inv = 1.0 / jnp.maximum(cnt, 1.0)
    h1 = jnp.maximum(agg * inv + r1_ref[...], 0.0)
    t2_ref[...] = _dg(h1, w2l_ref[...])
    r2_ref[...] = _dg(h1, w2r_ref[...]) + b2l_ref[...]


def _tc2(aggp, cntp, r1, W2l, b2l, W2r):
    R = 2000
    grid = (N // R,)
    full = lambda shape: pl.BlockSpec(shape, lambda i: (0,) * len(shape))
    return pl.pallas_call(
        _tc2_body,
        grid=grid,
        in_specs=[
            pl.BlockSpec((NC, R, H), lambda i: (0, i, 0)),
            pl.BlockSpec((NC, R, H), lambda i: (0, i, 0)),
            pl.BlockSpec((R, H), lambda i: (i, 0)),
            full((H, H)), full((1, H)), full((H, H)),
        ],
        out_specs=[pl.BlockSpec((R, H), lambda i: (i, 0))] * 2,
        out_shape=[jax.ShapeDtypeStruct((N, H), jnp.float32)] * 2,
    )(aggp, cntp, r1, W2l, b2l.reshape(1, H), W2r)


def _tc3_body(aggp_ref, cntp_ref, r2_ref, h2_ref):
    agg = aggp_ref[0] + aggp_ref[1]
    cnt = cntp_ref[0, :, 0:1] + cntp_ref[1, :, 0:1]
    inv = 1.0 / jnp.maximum(cnt, 1.0)
    h2_ref[...] = agg * inv + r2_ref[...]


def _tc3(aggp, cntp, r2):
    R = 2048
    grid = (NA // R,)
    return pl.pallas_call(
        _tc3_body,
        grid=grid,
        in_specs=[
            pl.BlockSpec((NC, R, H), lambda i: (0, i, 0)),
            pl.BlockSpec((NC, R, H), lambda i: (0, i, 0)),
            pl.BlockSpec((R, H), lambda i: (i, 0)),
        ],
        out_specs=pl.BlockSpec((R, H), lambda i: (i, 0)),
        out_shape=jax.ShapeDtypeStruct((NA, H), jnp.float32),
    )(aggp, cntp, r2)


def _seg_body(tbl, srcp, dstp, zh, aggp,
              sv0, sv1, sv2, sv3, dv0, dv1, dv2, dv3, rows0, rows1, aggs,
              si0, si1, si2, si3, sg0, sg1, ss0, ss1):
    c = lax.axis_index("c")
    s = lax.axis_index("s")
    srcv = [sv0, sv1, sv2, sv3]
    dstv = [dv0, dv1, dv2, dv3]
    rows = [rows0, rows1]
    sem_i = [si0, si1, si2, si3]
    sem_g = [sg0, sg1]
    sem_s = [ss0, ss1]

    r0 = s * STRIPE
    pltpu.sync_copy(zh.at[pl.ds(r0, STRIPE)], aggs.at[pl.ds(r0, STRIPE)])
    plsc.subcore_barrier()
    K = jnp.where(c == KSLOW_CORE, KSLOW, KFAST)
    nquad = jnp.where(c == KSLOW_CORE, KSLOW // 4, KFAST // 4)
    base_chunk = jnp.where(
        c == KSLOW_CORE,
        jnp.where(c == 0, s * KSLOW, NS * KFAST + s * KSLOW),
        jnp.where(c == 0, s * KFAST, NS * KSLOW + s * KFAST))
    base = base_chunk * CHUNK

    def issue_idx(k, q):
        pltpu.async_copy(srcp.at[pl.ds(base + k * CHUNK, CHUNK)],
                         srcv[q], sem_i[q])
        pltpu.async_copy(dstp.at[pl.ds(base + k * CHUNK, CHUNK)],
                         dstv[q], sem_i[q])

    def wait_idx(q):
        pltpu.make_async_copy(srcp.at[pl.ds(base, CHUNK)],
                              srcv[q], sem_i[q]).wait()
        pltpu.make_async_copy(dstp.at[pl.ds(base, CHUNK)],
                              dstv[q], sem_i[q]).wait()

    def wait_scat(b, q):
        pltpu.make_async_copy(rows[b], aggs.at[dstv[q]], sem_s[b]).wait()

    issue_idx(0, 0)
    issue_idx(1, 1)

    def quad(gg, _):
        for j in range(4):
            k = gg * 4 + j
            b = j % 2
            wait_idx(j)

            @pl.when(k >= 2)
            def _():
                wait_scat(b, (j + 2) % 4)

            @pl.when(k < K - 2)
            def _():
                issue_idx(k + 2, (j + 2) % 4)

            pltpu.async_copy(tbl.at[srcv[j]], rows[b], sem_g[b])
            pltpu.make_async_copy(tbl.at[srcv[j]], rows[b], sem_g[b]).wait()
            pltpu.async_copy(rows[b], aggs.at[dstv[j]], sem_s[b], add=True)
        return 0
    lax.fori_loop(0, nquad, quad, 0)
    wait_scat(0, 2)
    wait_scat(1, 3)
    plsc.subcore_barrier()

    pltpu.sync_copy(aggs.at[pl.ds(r0, STRIPE)], aggp.at[c, pl.ds(r0, STRIPE)])


_seg = pl.kernel(
    _seg_body,
    out_type=jax.ShapeDtypeStruct((NC, NA, H), jnp.float32),
    mesh=_mesh,
    scratch_types=(
        [pltpu.VMEM((CHUNK,), jnp.int32)] * 8 +
        [pltpu.VMEM((CHUNK, H), jnp.float32)] * 2 +
        [pltpu.VMEM_SHARED((NA, H), jnp.float32)] +
        [pltpu.SemaphoreType.DMA] * 8
    ),
)


def _cnt_body(dstp, zh, cntp, dstv, ones, cnts):
    c = lax.axis_index("c")
    s = lax.axis_index("s")
    wid = c * NS + s

    r0 = s * STRIPE
    pltpu.sync_copy(zh.at[pl.ds(r0, STRIPE)], cnts.at[pl.ds(r0, STRIPE)])

    def fill1(i, _):
        for j in range(H // 16):
            ones[i, pl.ds(j * 16, 16)] = jnp.ones((16,), jnp.float32)
        return 0
    lax.fori_loop(0, CHUNK, fill1, 0)
    plsc.subcore_barrier()
    base = wid * EPW

    def chunk(k, _):
        pltpu.sync_copy(dstp.at[pl.ds(base + k * CHUNK, CHUNK)], dstv)
        pltpu.sync_copy(ones, cnts.at[dstv], add=True)
        return 0
    lax.fori_loop(0, NCHUNK, chunk, 0)
    plsc.subcore_barrier()
    pltpu.sync_copy(cnts.at[pl.ds(r0, STRIPE)], cntp.at[c, pl.ds(r0, STRIPE)])


_cnt = pl.kernel(
    _cnt_body,
    out_type=jax.ShapeDtypeStruct((NC, NA, H), jnp.float32),
    mesh=_mesh,
    scratch_types=[
        pltpu.VMEM((CHUNK,), jnp.int32),
        pltpu.VMEM((CHUNK, H), jnp.float32),
        pltpu.VMEM_SHARED((NA, H), jnp.float32),
    ],
)


def _cls_body(tbl, headp, tailp, outp, hidx, tidx,
              hrows, trows, outv, tbls, sem_h, sem_t):
    c = lax.axis_index("c")
    s = lax.axis_index("s")
    wid = c * NS + s
    r0 = s * STRIPE
    pltpu.sync_copy(tbl.at[pl.ds(r0, STRIPE)], tbls.at[pl.ds(r0, STRIPE)])
    pltpu.sync_copy(headp.at[pl.ds(wid * ELW, ELW)], hidx)
    pltpu.sync_copy(tailp.at[pl.ds(wid * ELW, ELW)], tidx)
    plsc.subcore_barrier()

    lanes = lax.iota(jnp.int32, 16)

    def chunk(k, _):
        pltpu.async_copy(tbls.at[hidx.at[pl.ds(k * CHUNK, CHUNK)]],
                         hrows, sem_h)
        pltpu.async_copy(tbls.at[tidx.at[pl.ds(k * CHUNK, CHUNK)]],
                         trows, sem_t)
        pltpu.make_async_copy(tbls.at[hidx.at[pl.ds(0, CHUNK)]],
                              hrows, sem_h).wait()
        pltpu.make_async_copy(tbls.at[tidx.at[pl.ds(0, CHUNK)]],
                              trows, sem_t).wait()

        def group(g, _):
            def edge(i, res):
                e = g * 16 + i
                acc = hrows[e, pl.ds(0, 16)] * trows[e, pl.ds(0, 16)]
                for j in range(1, H // 16):
                    acc = acc + (hrows[e, pl.ds(j * 16, 16)] *
                                 trows[e, pl.ds(j * 16, 16)])
                return jnp.where(lanes == i, jnp.sum(acc), res)
            res = lax.fori_loop(0, 16, edge, jnp.zeros((16,), jnp.float32))
            outv[pl.ds(k * CHUNK + g * 16, 16)] = res
            return 0
        lax.fori_loop(0, CHUNK // 16, group, 0)
        return 0
    lax.fori_loop(0, NLCHUNK, chunk, 0)
    pltpu.sync_copy(outv, outp.at[pl.ds(wid * ELW, ELW)])


_cls = pl.kernel(
    _cls_body,
    out_type=jax.ShapeDtypeStruct((ELP,), jnp.float32),
    mesh=_mesh,
    compiler_params=pltpu.CompilerParams(needs_layout_passes=False),
    scratch_types=(
        [pltpu.VMEM((ELW,), jnp.int32)] * 2 +
        [pltpu.VMEM((CHUNK, H), jnp.float32)] * 2 +
        [pltpu.VMEM((ELW,), jnp.float32)] +
        [pltpu.VMEM_SHARED((NA, H), jnp.float32)] +
        [pltpu.SemaphoreType.DMA] * 2
    ),
)


def kernel(x, edge_index, edge_label_index, W_lin, b_lin,
           W1l, b1l, W1r, W2l, b2l, W2r):
    src = jnp.concatenate([edge_index[0], jnp.zeros((EP - E,), jnp.int32)])
    dst = jnp.concatenate([edge_index[1], jnp.full((EP - E,), N, jnp.int32)])
    head = jnp.concatenate(
        [edge_label_index[0], jnp.zeros((ELP - EL,), jnp.int32)])
    tail = jnp.concatenate(
        [edge_label_index[1], jnp.zeros((ELP - EL,), jnp.int32)])
    zh = jnp.zeros((NA, H), jnp.float32)

    cntp = _cnt(dst, zh)
    t1, r1 = _tc1(x, W_lin, b_lin, W1l, b1l, W1r)
    aggp1 = _seg(t1, src, dst, zh)
    t2, r2 = _tc2(aggp1, cntp, r1, W2l, b2l, W2r)
    aggp2 = _seg(t2, src, dst, zh)
    h2 = _tc3(aggp2, cntp, r2)
    return _cls(h2, head, tail)[:EL]


# speedup vs baseline: 1.1404x; 1.0004x over previous
import functools

import jax
import jax.numpy as jnp
from jax import lax
from jax.experimental import pallas as pl
from jax.experimental.pallas import tpu as pltpu
from jax.experimental.pallas import tpu_sc as plsc

N = 10000
E = 320000
EL = 100000
D_IN = 384
H = 128

NC = 2
NS = 16
NW = NC * NS

CHUNK = 128
EPW = 10240
EP = NW * EPW
NCHUNK = EPW // CHUNK
KSLOW_CORE = 1
KSLOW = 48
KFAST = 2 * NCHUNK - KSLOW
NA = 10240
STRIPE = NA // NS

ELW = 3328
ELP = NW * ELW
NLCHUNK = ELW // CHUNK

_mesh = plsc.VectorSubcoreMesh(core_axis_name="c", subcore_axis_name="s")


def _dg(a, b):
    return lax.dot_general(a, b, (((1,), (1,)), ((), ())),
                           preferred_element_type=jnp.float32)


def _tc1_body(x_ref, wl_ref, bl_ref, w1l_ref, b1l_ref, w1r_ref,
              t1_ref, r1_ref):
    h0 = _dg(x_ref[...], wl_ref[...]) + bl_ref[...]
    t1_ref[...] = _dg(h0, w1l_ref[...])
    r1_ref[...] = _dg(h0, w1r_ref[...]) + b1l_ref[...]


def _tc1(x, W_lin, b_lin, W1l, b1l, W1r):
    R = 2000
    grid = (N // R,)
    full = lambda shape: pl.BlockSpec(shape, lambda i: (0,) * len(shape))
    return pl.pallas_call(
        _tc1_body,
        grid=grid,
        in_specs=[
            pl.BlockSpec((R, D_IN), lambda i: (i, 0)),
            full((H, D_IN)), full((1, H)), full((H, H)), full((1, H)),
            full((H, H)),
        ],
        out_specs=[pl.BlockSpec((R, H), lambda i: (i, 0))] * 2,
        out_shape=[jax.ShapeDtypeStruct((N, H), jnp.float32)] * 2,
    )(x, W_lin, b_lin.reshape(1, H), W1l, b1l.reshape(1, H), W1r)


def _tc2_body(aggp_ref, cntp_ref, r1_ref, w2l_ref, b2l_ref, w2r_ref,
              t2_ref, r2_ref):
    agg = aggp_ref[0] + aggp_ref[1]
    cnt = cntp_ref[0, :, 0:1] + cntp_ref[1, :, 0:1]
    inv = 1.0 / jnp.maximum(cnt, 1.0)
    h1 = jnp.maximum(agg * inv + r1_ref[...], 0.0)
    t2_ref[...] = _dg(h1, w2l_ref[...])
    r2_ref[...] = _dg(h1, w2r_ref[...]) + b2l_ref[...]


def _tc2(aggp, cntp, r1, W2l, b2l, W2r):
    R = 2000
    grid = (N // R,)
    full = lambda shape: pl.BlockSpec(shape, lambda i: (0,) * len(shape))
    return pl.pallas_call(
        _tc2_body,
        grid=grid,
        in_specs=[
            pl.BlockSpec((NC, R, H), lambda i: (0, i, 0)),
            pl.BlockSpec((NC, R, H), lambda i: (0, i, 0)),
            pl.BlockSpec((R, H), lambda i: (i, 0)),
            full((H, H)), full((1, H)), full((H, H)),
        ],
        out_specs=[pl.BlockSpec((R, H), lambda i: (i, 0))] * 2,
        out_shape=[jax.ShapeDtypeStruct((N, H), jnp.float32)] * 2,
    )(aggp, cntp, r1, W2l, b2l.reshape(1, H), W2r)


def _tc3_body(aggp_ref, cntp_ref, r2_ref, h2_ref):
    agg = aggp_ref[0] + aggp_ref[1]
    cnt = cntp_ref[0, :, 0:1] + cntp_ref[1, :, 0:1]
    inv = 1.0 / jnp.maximum(cnt, 1.0)
    h2_ref[...] = agg * inv + r2_ref[...]


def _tc3(aggp, cntp, r2):
    R = 2048
    grid = (NA // R,)
    return pl.pallas_call(
        _tc3_body,
        grid=grid,
        in_specs=[
            pl.BlockSpec((NC, R, H), lambda i: (0, i, 0)),
            pl.BlockSpec((NC, R, H), lambda i: (0, i, 0)),
            pl.BlockSpec((R, H), lambda i: (i, 0)),
        ],
        out_specs=pl.BlockSpec((R, H), lambda i: (i, 0)),
        out_shape=jax.ShapeDtypeStruct((NA, H), jnp.float32),
    )(aggp, cntp, r2)


def _seg_body(tbl, srcp, dstp, zh, aggp,
              sv0, sv1, sv2, sv3, dv0, dv1, dv2, dv3, rows0, rows1, aggs,
              si0, si1, si2, si3, sg0, sg1, ss0, ss1):
    c = lax.axis_index("c")
    s = lax.axis_index("s")
    srcv = [sv0, sv1, sv2, sv3]
    dstv = [dv0, dv1, dv2, dv3]
    rows = [rows0, rows1]
    sem_i = [si0, si1, si2, si3]
    sem_g = [sg0, sg1]
    sem_s = [ss0, ss1]

    r0 = s * STRIPE
    pltpu.sync_copy(zh.at[pl.ds(r0, STRIPE)], aggs.at[pl.ds(r0, STRIPE)])
    plsc.subcore_barrier()
    K = jnp.where(c == KSLOW_CORE, KSLOW, KFAST)
    nquad = jnp.where(c == KSLOW_CORE, KSLOW // 4, KFAST // 4)
    base_chunk = jnp.where(
        c == KSLOW_CORE,
        jnp.where(c == 0, s * KSLOW, NS * KFAST + s * KSLOW),
        jnp.where(c == 0, s * KFAST, NS * KSLOW + s * KFAST))
    base = base_chunk * CHUNK

    def issue_idx(k, q):
        pltpu.async_copy(srcp.at[pl.ds(base + k * CHUNK, CHUNK)],
                         srcv[q], sem_i[q])
        pltpu.async_copy(dstp.at[pl.ds(base + k * CHUNK, CHUNK)],
                         dstv[q], sem_i[q])

    def wait_idx(q):
        pltpu.make_async_copy(srcp.at[pl.ds(base, CHUNK)],
                              srcv[q], sem_i[q]).wait()
        pltpu.make_async_copy(dstp.at[pl.ds(base, CHUNK)],
                              dstv[q], sem_i[q]).wait()

    def wait_scat(b, q):
        pltpu.make_async_copy(rows[b], aggs.at[dstv[q]], sem_s[b]).wait()

    def wait_gath(b, q):
        pltpu.make_async_copy(tbl.at[srcv[q]], rows[b], sem_g[b]).wait()

    issue_idx(0, 0)
    issue_idx(1, 1)
    wait_idx(0)
    pltpu.async_copy(tbl.at[srcv[0]], rows[0], sem_g[0])

    def quad(gg, _):
        for j in range(4):
            k = gg * 4 + j
            b = j % 2
            jn = (j + 1) % 4

            @pl.when(k + 1 < K)
            def _():
                wait_idx(jn)

                @pl.when(k >= 1)
                def _():
                    wait_scat(1 - b, (j + 3) % 4)
                pltpu.async_copy(tbl.at[srcv[jn]], rows[1 - b],
                                 sem_g[1 - b])

            @pl.when(k < K - 2)
            def _():
                issue_idx(k + 2, (j + 2) % 4)

            wait_gath(b, j)
            pltpu.async_copy(rows[b], aggs.at[dstv[j]], sem_s[b], add=True)
        return 0
    lax.fori_loop(0, nquad, quad, 0)
    wait_scat(0, 2)
    wait_scat(1, 3)
    plsc.subcore_barrier()

    pltpu.sync_copy(aggs.at[pl.ds(r0, STRIPE)], aggp.at[c, pl.ds(r0, STRIPE)])


_seg = pl.kernel(
    _seg_body,
    out_type=jax.ShapeDtypeStruct((NC, NA, H), jnp.float32),
    mesh=_mesh,
    scratch_types=(
        [pltpu.VMEM((CHUNK,), jnp.int32)] * 8 +
        [pltpu.VMEM((CHUNK, H), jnp.float32)] * 2 +
        [pltpu.VMEM_SHARED((NA, H), jnp.float32)] +
        [pltpu.SemaphoreType.DMA] * 8
    ),
)


def _cnt_body(dstp, zh, cntp, dstv, ones, cnts):
    c = lax.axis_index("c")
    s = lax.axis_index("s")
    wid = c * NS + s

    r0 = s * STRIPE
    pltpu.sync_copy(zh.at[pl.ds(r0, STRIPE)], cnts.at[pl.ds(r0, STRIPE)])

    def fill1(i, _):
        for j in range(H // 16):
            ones[i, pl.ds(j * 16, 16)] = jnp.ones((16,), jnp.float32)
        return 0
    lax.fori_loop(0, CHUNK, fill1, 0)
    plsc.subcore_barrier()
    base = wid * EPW

    def chunk(k, _):
        pltpu.sync_copy(dstp.at[pl.ds(base + k * CHUNK, CHUNK)], dstv)
        pltpu.sync_copy(ones, cnts.at[dstv], add=True)
        return 0
    lax.fori_loop(0, NCHUNK, chunk, 0)
    plsc.subcore_barrier()
    pltpu.sync_copy(cnts.at[pl.ds(r0, STRIPE)], cntp.at[c, pl.ds(r0, STRIPE)])


_cnt = pl.kernel(
    _cnt_body,
    out_type=jax.ShapeDtypeStruct((NC, NA, H), jnp.float32),
    mesh=_mesh,
    scratch_types=[
        pltpu.VMEM((CHUNK,), jnp.int32),
        pltpu.VMEM((CHUNK, H), jnp.float32),
        pltpu.VMEM_SHARED((NA, H), jnp.float32),
    ],
)


def _cls_body(tbl, headp, tailp, outp, hidx, tidx,
              hrows, trows, outv, tbls, sem_h, sem_t):
    c = lax.axis_index("c")
    s = lax.axis_index("s")
    wid = c * NS + s
    r0 = s * STRIPE
    pltpu.sync_copy(tbl.at[pl.ds(r0, STRIPE)], tbls.at[pl.ds(r0, STRIPE)])
    pltpu.sync_copy(headp.at[pl.ds(wid * ELW, ELW)], hidx)
    pltpu.sync_copy(tailp.at[pl.ds(wid * ELW, ELW)], tidx)
    plsc.subcore_barrier()

    lanes = lax.iota(jnp.int32, 16)

    def chunk(k, _):
        pltpu.async_copy(tbls.at[hidx.at[pl.ds(k * CHUNK, CHUNK)]],
                         hrows, sem_h)
        pltpu.async_copy(tbls.at[tidx.at[pl.ds(k * CHUNK, CHUNK)]],
                         trows, sem_t)
        pltpu.make_async_copy(tbls.at[hidx.at[pl.ds(0, CHUNK)]],
                              hrows, sem_h).wait()
        pltpu.make_async_copy(tbls.at[tidx.at[pl.ds(0, CHUNK)]],
                              trows, sem_t).wait()

        def group(g, _):
            def edge(i, res):
                e = g * 16 + i
                acc = hrows[e, pl.ds(0, 16)] * trows[e, pl.ds(0, 16)]
                for j in range(1, H // 16):
                    acc = acc + (hrows[e, pl.ds(j * 16, 16)] *
                                 trows[e, pl.ds(j * 16, 16)])
                return jnp.where(lanes == i, jnp.sum(acc), res)
            res = lax.fori_loop(0, 16, edge, jnp.zeros((16,), jnp.float32))
            outv[pl.ds(k * CHUNK + g * 16, 16)] = res
            return 0
        lax.fori_loop(0, CHUNK // 16, group, 0)
        return 0
    lax.fori_loop(0, NLCHUNK, chunk, 0)
    pltpu.sync_copy(outv, outp.at[pl.ds(wid * ELW, ELW)])


_cls = pl.kernel(
    _cls_body,
    out_type=jax.ShapeDtypeStruct((ELP,), jnp.float32),
    mesh=_mesh,
    compiler_params=pltpu.CompilerParams(needs_layout_passes=False),
    scratch_types=(
        [pltpu.VMEM((ELW,), jnp.int32)] * 2 +
        [pltpu.VMEM((CHUNK, H), jnp.float32)] * 2 +
        [pltpu.VMEM((ELW,), jnp.float32)] +
        [pltpu.VMEM_SHARED((NA, H), jnp.float32)] +
        [pltpu.SemaphoreType.DMA] * 2
    ),
)


def kernel(x, edge_index, edge_label_index, W_lin, b_lin,
           W1l, b1l, W1r, W2l, b2l, W2r):
    src = jnp.concatenate([edge_index[0], jnp.zeros((EP - E,), jnp.int32)])
    dst = jnp.concatenate([edge_index[1], jnp.full((EP - E,), N, jnp.int32)])
    head = jnp.concatenate(
        [edge_label_index[0], jnp.zeros((ELP - EL,), jnp.int32)])
    tail = jnp.concatenate(
        [edge_label_index[1], jnp.zeros((ELP - EL,), jnp.int32)])
    zh = jnp.zeros((NA, H), jnp.float32)

    cntp = _cnt(dst, zh)
    t1, r1 = _tc1(x, W_lin, b_lin, W1l, b1l, W1r)
    aggp1 = _seg(t1, src, dst, zh)
    t2, r2 = _tc2(aggp1, cntp, r1, W2l, b2l, W2r)
    aggp2 = _seg(t2, src, dst, zh)
    h2 = _tc3(aggp2, cntp, r2)
    return _cls(h2, head, tail)[:EL]
